# baseline (speedup 1.0000x reference)
import functools

import jax
import jax.numpy as jnp
from jax import lax
from jax.experimental import pallas as pl
from jax.experimental.pallas import tpu as pltpu
from jax.experimental.pallas import tpu_sc as plsc

B = 16384
NZ = 1024
L = 16
NC = 2
NS = 16
NW = NC * NS
ROWS_PER_W = B // NW
R = 32
C = ROWS_PER_W // R
NJ = NZ // L


def _sc_body(h_hbm, idx_hbm, out_hbm, idx_v, in_v, out_v):
    wid = lax.axis_index("s") * NC + lax.axis_index("c")
    row0 = wid * ROWS_PER_W
    pltpu.sync_copy(idx_hbm, idx_v)

    def chunk_body(g):
        base = row0 + g * R
        pltpu.sync_copy(h_hbm.at[pl.ds(base, R), :], in_v)

        def row_body(r):
            rvec = jnp.broadcast_to(r, (L,)).astype(jnp.int32)
            for j in range(NJ):
                cidx = idx_v[pl.ds(j * L, L)]
                g16 = plsc.load_gather(in_v, [rvec, cidx])
                out_v[r, pl.ds(j * L, L)] = g16

        lax.fori_loop(0, R, lambda r, c: (row_body(r), c)[1], 0)
        pltpu.sync_copy(out_v, out_hbm.at[pl.ds(base, R), :])

    lax.fori_loop(0, C, lambda g, c: (chunk_body(g), c)[1], 0)


def kernel(h, indices):
    mesh = plsc.VectorSubcoreMesh(core_axis_name="c", subcore_axis_name="s")
    k = pl.kernel(
        _sc_body,
        out_type=jax.ShapeDtypeStruct((B, NZ), jnp.float32),
        mesh=mesh,
        scratch_types=[
            pltpu.VMEM((NZ,), jnp.int32),
            pltpu.VMEM((R, NZ), jnp.float32),
            pltpu.VMEM((R, NZ), jnp.float32),
        ],
        compiler_params=pltpu.CompilerParams(use_tc_tiling_on_sc=False,
                                             needs_layout_passes=False),
    )
    return k(h, indices)


# speedup vs baseline: 2.0738x; 2.0738x over previous
import functools

import jax
import jax.numpy as jnp
from jax import lax
from jax.experimental import pallas as pl
from jax.experimental.pallas import tpu as pltpu
from jax.experimental.pallas import tpu_sc as plsc

B = 16384
NZ = 1024
L = 16
NC = 2
NS = 16
NW = NC * NS
ROWS_PER_W = B // NW
R = 32
C = ROWS_PER_W // R
NJ = NZ // L


def _sc_body(h_hbm, idx_hbm, out_hbm, idx_v, in_v, out_v):
    wid = lax.axis_index("s") * NC + lax.axis_index("c")
    row0 = wid * ROWS_PER_W
    pltpu.sync_copy(idx_hbm, idx_v)

    def chunk_body(g):
        base = row0 + g * R
        pltpu.sync_copy(h_hbm.at[pl.ds(base, R), :], in_v)

        for j in range(NJ):
            cidx = idx_v[pl.ds(j * L, L)]

            @plsc.parallel_loop(0, R, step=1, unroll=8)
            def _rb(r):
                rvec = jnp.broadcast_to(r, (L,)).astype(jnp.int32)
                out_v[r, pl.ds(j * L, L)] = plsc.load_gather(
                    in_v, [rvec, cidx])

        pltpu.sync_copy(out_v, out_hbm.at[pl.ds(base, R), :])

    lax.fori_loop(0, C, lambda g, c: (chunk_body(g), c)[1], 0)


def kernel(h, indices):
    mesh = plsc.VectorSubcoreMesh(core_axis_name="c", subcore_axis_name="s")
    k = pl.kernel(
        _sc_body,
        out_type=jax.ShapeDtypeStruct((B, NZ), jnp.float32),
        mesh=mesh,
        scratch_types=[
            pltpu.VMEM((NZ,), jnp.int32),
            pltpu.VMEM((R, NZ), jnp.float32),
            pltpu.VMEM((R, NZ), jnp.float32),
        ],
        compiler_params=pltpu.CompilerParams(use_tc_tiling_on_sc=False,
                                             needs_layout_passes=False),
    )
    return k(h, indices)


# speedup vs baseline: 2.5152x; 1.2128x over previous
import functools

import jax
import jax.numpy as jnp
from jax import lax
from jax.experimental import pallas as pl
from jax.experimental.pallas import tpu as pltpu
from jax.experimental.pallas import tpu_sc as plsc

B = 16384
NZ = 1024
L = 16
NC = 2
NS = 16
NW = NC * NS
ROWS_PER_W = B // NW
R = 32
C = ROWS_PER_W // R
NJ = NZ // L


def _sc_body(h_hbm, idx_hbm, out_hbm, idx_v, in_v, out_v):
    wid = lax.axis_index("s") * NC + lax.axis_index("c")
    row0 = wid * ROWS_PER_W
    pltpu.sync_copy(idx_hbm, idx_v)

    def chunk_body(g):
        base = row0 + g * R
        pltpu.sync_copy(h_hbm.at[pl.ds(base, R), :], in_v)

        @plsc.parallel_loop(0, NJ, step=1)
        def _jb(j):
            cidx = idx_v[pl.ds(j * L, L)]
            for r in range(R):
                rvec = jnp.full((L,), r, jnp.int32)
                out_v[r, pl.ds(j * L, L)] = plsc.load_gather(
                    in_v, [rvec, cidx])

        pltpu.sync_copy(out_v, out_hbm.at[pl.ds(base, R), :])

    lax.fori_loop(0, C, lambda g, c: (chunk_body(g), c)[1], 0)


def kernel(h, indices):
    mesh = plsc.VectorSubcoreMesh(core_axis_name="c", subcore_axis_name="s")
    k = pl.kernel(
        _sc_body,
        out_type=jax.ShapeDtypeStruct((B, NZ), jnp.float32),
        mesh=mesh,
        scratch_types=[
            pltpu.VMEM((NZ,), jnp.int32),
            pltpu.VMEM((R, NZ), jnp.float32),
            pltpu.VMEM((R, NZ), jnp.float32),
        ],
        compiler_params=pltpu.CompilerParams(use_tc_tiling_on_sc=False,
                                             needs_layout_passes=False),
    )
    return k(h, indices)
